# baseline (speedup 1.0000x reference)
import functools

import jax
import jax.numpy as jnp
from jax import lax
from jax.experimental import pallas as pl
from jax.experimental.pallas import tpu as pltpu
from jax.experimental.pallas import tpu_sc as plsc

_BH = 64
_K = 128
_NC = 2
_NS = 16
_NW = _NC * _NS
_CB = 8


def _sums_body(f_ref, g_ref, sc_ref, st_ref, cc_ref):
    h = pl.program_id(1)
    x = f_ref[0]
    m = (g_ref[0, 0] == 1).astype(jnp.float32)
    sc = jnp.sum(x * m[None], axis=(1, 2))
    st = jnp.sum(x, axis=(1, 2))
    cc = jnp.sum(m)
    C = sc.shape[0]
    scb = jnp.broadcast_to(sc[None, :], (8, C))
    stb = jnp.broadcast_to(st[None, :], (8, C))
    ccb = jnp.full((8, C), cc, jnp.float32)

    @pl.when(h == 0)
    def _init():
        sc_ref[0] = scb
        st_ref[0] = stb
        cc_ref[0] = ccb

    @pl.when(h != 0)
    def _acc():
        sc_ref[0] += scb
        st_ref[0] += stb
        cc_ref[0] += ccb


def _sc_sums_body(f_hbm, g_hbm, osc_hbm, ost_hbm, occ_hbm,
                  mi_v, mf_v, x_v, asc_v, ast_v, acc_v):
    B, C, H, W = f_hbm.shape
    rpw = _K // _NW
    nvr = W // 16
    wid = lax.axis_index("s") * _NC + lax.axis_index("c")
    r0 = (H - _K) + wid * rpw

    for b in range(B):
        pltpu.sync_copy(g_hbm.at[b, 0, pl.ds(r0, rpw), :], mi_v)
        cc16 = jnp.zeros((16,), jnp.float32)
        for r in range(rpw):
            def _mrow(j, cc, r=r):
                mi = mi_v[r, pl.ds(j * 16, 16)]
                mf = jnp.where(mi == 1, 1.0, 0.0).astype(jnp.float32)
                mf_v[r, pl.ds(j * 16, 16)] = mf
                return cc + mf
            cc16 = lax.fori_loop(0, nvr, _mrow, cc16)
        acc_v[0] = cc16
        pltpu.sync_copy(acc_v, occ_hbm.at[b, wid])

        def _chunk(k, _, b=b):
            pltpu.sync_copy(f_hbm.at[b, pl.ds(k * _CB, _CB), pl.ds(r0, rpw), :],
                            x_v)
            for cl in range(_CB):
                sc16 = jnp.zeros((16,), jnp.float32)
                st16 = jnp.zeros((16,), jnp.float32)
                for r in range(rpw):
                    def _vrow(j, carry, cl=cl, r=r):
                        s, t = carry
                        xi = x_v[cl, r, pl.ds(j * 16, 16)]
                        mf = mf_v[r, pl.ds(j * 16, 16)]
                        return (s + xi * mf, t + xi)
                    sc16, st16 = lax.fori_loop(0, nvr, _vrow, (sc16, st16))
                asc_v[k * _CB + cl] = sc16
                ast_v[k * _CB + cl] = st16
            return 0

        lax.fori_loop(0, C // _CB, _chunk, 0)
        pltpu.sync_copy(asc_v, osc_hbm.at[b, wid])
        pltpu.sync_copy(ast_v, ost_hbm.at[b, wid])


def _sc_sums(feature_map, ground_truth):
    B, C, H, W = feature_map.shape
    rpw = _K // _NW
    mesh = plsc.VectorSubcoreMesh(core_axis_name="c", subcore_axis_name="s")
    fn = functools.partial(
        pl.kernel,
        out_type=[
            jax.ShapeDtypeStruct((B, _NW, C, 16), jnp.float32),
            jax.ShapeDtypeStruct((B, _NW, C, 16), jnp.float32),
            jax.ShapeDtypeStruct((B, _NW, 1, 16), jnp.float32),
        ],
        mesh=mesh,
        scratch_types=[
            pltpu.VMEM((rpw, W), jnp.int32),
            pltpu.VMEM((rpw, W), jnp.float32),
            pltpu.VMEM((_CB, rpw, W), jnp.float32),
            pltpu.VMEM((C, 16), jnp.float32),
            pltpu.VMEM((C, 16), jnp.float32),
            pltpu.VMEM((1, 16), jnp.float32),
        ],
    )(_sc_sums_body)
    return fn(feature_map, ground_truth)


def _out_body(f_ref, g_ref, d_ref, bias_ref, o_ref):
    b = pl.program_id(0)
    x = f_ref[0]
    g = g_ref[0, 0]
    dv = d_ref[b]
    dot = jnp.sum(x * dv[:, None, None], axis=0)
    ss = jnp.sum(x * x, axis=0)
    nrm = jnp.maximum(jnp.sqrt(ss), 1e-12)
    z = dot / nrm
    z = jnp.where(g == 1, z, -z)
    o_ref[0] = jnp.exp(z + bias_ref[b, 0])


def kernel(feature_map, ground_truth):
    B, C, H, W = feature_map.shape
    nH_tc = (H - _K) // _BH

    sc_p, st_p, cc_p = pl.pallas_call(
        _sums_body,
        grid=(B, nH_tc),
        in_specs=[
            pl.BlockSpec((1, C, _BH, W), lambda b, h: (b, 0, h, 0)),
            pl.BlockSpec((1, 1, _BH, W), lambda b, h: (b, 0, h, 0)),
        ],
        out_specs=[
            pl.BlockSpec((1, 8, C), lambda b, h: (b, 0, 0)),
            pl.BlockSpec((1, 8, C), lambda b, h: (b, 0, 0)),
            pl.BlockSpec((1, 8, C), lambda b, h: (b, 0, 0)),
        ],
        out_shape=[
            jax.ShapeDtypeStruct((B, 8, C), jnp.float32),
            jax.ShapeDtypeStruct((B, 8, C), jnp.float32),
            jax.ShapeDtypeStruct((B, 8, C), jnp.float32),
        ],
        compiler_params=pltpu.CompilerParams(
            dimension_semantics=("parallel", "arbitrary"),
        ),
    )(feature_map, ground_truth)

    sc_sc, sc_st, sc_cc = _sc_sums(feature_map, ground_truth)

    sum_c = sc_p[:, 0, :] + sc_sc.sum(axis=(1, 3))
    sum_t = st_p[:, 0, :] + sc_st.sum(axis=(1, 3))
    cnt_c = cc_p[:, 0, 0] + sc_cc.sum(axis=(1, 2, 3))
    cnt_n = H * W - cnt_c
    sum_n = sum_t - sum_c
    valid = (cnt_c > 0) & (cnt_n > 0)
    pc = sum_c / jnp.maximum(cnt_c, 1.0)[:, None]
    pn = sum_n / jnp.maximum(cnt_n, 1.0)[:, None]
    pcn = pc / jnp.maximum(jnp.linalg.norm(pc, axis=1, keepdims=True), 1e-12)
    pnn = pn / jnp.maximum(jnp.linalg.norm(pn, axis=1, keepdims=True), 1e-12)
    d = pcn - pnn
    bias = jnp.where(valid, 0.0, -jnp.inf).astype(jnp.float32)
    bias_v = jnp.broadcast_to(bias[:, None], (B, C))

    out = pl.pallas_call(
        _out_body,
        grid=(B, H // _BH),
        in_specs=[
            pl.BlockSpec((1, C, _BH, W), lambda b, h: (b, 0, h, 0)),
            pl.BlockSpec((1, 1, _BH, W), lambda b, h: (b, 0, h, 0)),
            pl.BlockSpec((B, C), lambda b, h: (0, 0)),
            pl.BlockSpec((B, C), lambda b, h: (0, 0)),
        ],
        out_specs=pl.BlockSpec((1, _BH, W), lambda b, h: (b, h, 0)),
        out_shape=jax.ShapeDtypeStruct((B, H, W), jnp.float32),
        compiler_params=pltpu.CompilerParams(
            dimension_semantics=("parallel", "parallel"),
        ),
    )(feature_map, ground_truth, d, bias_v)

    return out


# speedup vs baseline: 1.6597x; 1.6597x over previous
import jax
import jax.numpy as jnp
from jax.experimental import pallas as pl
from jax.experimental.pallas import tpu as pltpu

_BH = 64


def _sums_body(f_ref, g_ref, sc_ref, st_ref, cc_ref):
    h = pl.program_id(1)
    x = f_ref[0]
    m = (g_ref[0, 0] == 1).astype(jnp.float32)
    sc = jnp.sum(x * m[None], axis=(1, 2))
    st = jnp.sum(x, axis=(1, 2))
    cc = jnp.sum(m)
    C = sc.shape[0]
    scb = jnp.broadcast_to(sc[None, :], (8, C))
    stb = jnp.broadcast_to(st[None, :], (8, C))
    ccb = jnp.full((8, C), cc, jnp.float32)

    @pl.when(h == 0)
    def _init():
        sc_ref[0] = scb
        st_ref[0] = stb
        cc_ref[0] = ccb

    @pl.when(h != 0)
    def _acc():
        sc_ref[0] += scb
        st_ref[0] += stb
        cc_ref[0] += ccb


def _out_body(f_ref, g_ref, d_ref, bias_ref, o_ref):
    b = pl.program_id(0)
    x = f_ref[0]
    g = g_ref[0, 0]
    dv = d_ref[b]
    dot = jnp.sum(x * dv[:, None, None], axis=0)
    ss = jnp.sum(x * x, axis=0)
    nrm = jnp.maximum(jnp.sqrt(ss), 1e-12)
    z = dot / nrm
    z = jnp.where(g == 1, z, -z)
    o_ref[0] = jnp.exp(z + bias_ref[b, 0])


def kernel(feature_map, ground_truth):
    B, C, H, W = feature_map.shape
    nH = H // _BH

    sc_p, st_p, cc_p = pl.pallas_call(
        _sums_body,
        grid=(B, nH),
        in_specs=[
            pl.BlockSpec((1, C, _BH, W), lambda b, h: (b, 0, h, 0)),
            pl.BlockSpec((1, 1, _BH, W), lambda b, h: (b, 0, h, 0)),
        ],
        out_specs=[
            pl.BlockSpec((1, 8, C), lambda b, h: (b, 0, 0)),
            pl.BlockSpec((1, 8, C), lambda b, h: (b, 0, 0)),
            pl.BlockSpec((1, 8, C), lambda b, h: (b, 0, 0)),
        ],
        out_shape=[
            jax.ShapeDtypeStruct((B, 8, C), jnp.float32),
            jax.ShapeDtypeStruct((B, 8, C), jnp.float32),
            jax.ShapeDtypeStruct((B, 8, C), jnp.float32),
        ],
        compiler_params=pltpu.CompilerParams(
            dimension_semantics=("parallel", "arbitrary"),
        ),
    )(feature_map, ground_truth)

    sum_c = sc_p[:, 0, :]
    sum_t = st_p[:, 0, :]
    cnt_c = cc_p[:, 0, 0]
    cnt_n = H * W - cnt_c
    sum_n = sum_t - sum_c
    valid = (cnt_c > 0) & (cnt_n > 0)
    pc = sum_c / jnp.maximum(cnt_c, 1.0)[:, None]
    pn = sum_n / jnp.maximum(cnt_n, 1.0)[:, None]
    pcn = pc / jnp.maximum(jnp.linalg.norm(pc, axis=1, keepdims=True), 1e-12)
    pnn = pn / jnp.maximum(jnp.linalg.norm(pn, axis=1, keepdims=True), 1e-12)
    d = pcn - pnn
    bias = jnp.where(valid, 0.0, -jnp.inf).astype(jnp.float32)
    bias_v = jnp.broadcast_to(bias[:, None], (B, C))

    out = pl.pallas_call(
        _out_body,
        grid=(B, nH),
        in_specs=[
            pl.BlockSpec((1, C, _BH, W), lambda b, h: (b, 0, h, 0)),
            pl.BlockSpec((1, 1, _BH, W), lambda b, h: (b, 0, h, 0)),
            pl.BlockSpec((B, C), lambda b, h: (0, 0)),
            pl.BlockSpec((B, C), lambda b, h: (0, 0)),
        ],
        out_specs=pl.BlockSpec((1, _BH, W), lambda b, h: (b, h, 0)),
        out_shape=jax.ShapeDtypeStruct((B, H, W), jnp.float32),
        compiler_params=pltpu.CompilerParams(
            dimension_semantics=("parallel", "parallel"),
        ),
    )(feature_map, ground_truth, d, bias_v)

    return out
